# baseline (speedup 1.0000x reference)
import functools
import jax
import jax.numpy as jnp
from jax import lax
from jax.experimental import pallas as pl
from jax.experimental.pallas import tpu as pltpu
from jax.experimental.pallas import tpu_sc as plsc

_NC = 2
_NS = 16
_NW = _NC * _NS
_NBUF = 4
_TR_CHUNK = 4096


def _tr_body(in_ref, out_ref):
    t = jnp.swapaxes(in_ref[...], 0, 1)
    tb = t.astype(jnp.bfloat16)
    out_ref[:, 0:tb.shape[1]] = tb


def _make_pool(batch, seq, emb, tw):
    assert batch % _NW == 0
    b_per_w = batch // _NW
    c0 = 128 if seq > 128 else seq
    c1 = seq - c0
    assert c0 % 8 == 0 and c1 % 8 == 0 and 0 < c1 <= 128
    mesh = plsc.VectorSubcoreMesh(
        core_axis_name="c", subcore_axis_name="s",
        num_cores=_NC, num_subcores=_NS)

    @functools.partial(
        pl.kernel,
        out_type=jax.ShapeDtypeStruct((batch, emb), jnp.float32),
        mesh=mesh,
        scratch_types=[
            pltpu.VMEM((b_per_w, seq), jnp.int32),
            pltpu.VMEM((_NBUF, seq, tw), jnp.bfloat16),
            pltpu.VMEM((b_per_w, emb), jnp.float32),
        ] + [pltpu.SemaphoreType.DMA] * _NBUF,
        compiler_params=pltpu.CompilerParams(
            use_tc_tiling_on_sc=False, needs_layout_passes=False),
    )
    def pool(x_hbm, emb_hbm, out_hbm, idx_v, rows_v, acc_v, *sems):
        wid = lax.axis_index("s") * _NC + lax.axis_index("c")
        base = wid * b_per_w
        pltpu.sync_copy(x_hbm.at[pl.ds(base, b_per_w)], idx_v)
        scale = jnp.float32(1.0 / seq)
        himask = jnp.full((16,), -65536, dtype=jnp.int32)

        def issue(row, slot):
            pltpu.async_copy(
                emb_hbm.at[idx_v.at[row, pl.ds(0, c0)]],
                rows_v.at[slot, pl.ds(0, c0)], sems[slot])
            pltpu.async_copy(
                emb_hbm.at[idx_v.at[row, pl.ds(c0, c1)]],
                rows_v.at[slot, pl.ds(c0, c1)], sems[slot])

        def wait_slot(slot):
            pltpu.make_async_copy(
                emb_hbm.at[idx_v.at[0, pl.ds(0, c0)]],
                rows_v.at[slot, pl.ds(0, c0)], sems[slot]).wait()
            pltpu.make_async_copy(
                emb_hbm.at[idx_v.at[0, pl.ds(c0, c1)]],
                rows_v.at[slot, pl.ds(c0, c1)], sems[slot]).wait()

        for s in range(_NBUF):
            issue(jnp.int32(s), s)

        nacc = emb // 16

        def outer(g_idx, carry):
            g = g_idx * _NBUF
            for s in range(_NBUF):
                row = g + s
                wait_slot(s)

                def red8(i, accs):
                    r0 = i * 8
                    new = list(accs)
                    for r in range(8):
                        for c in range(emb // 32):
                            b = rows_v[s, r0 + r, pl.ds(32 * c, 32)]
                            w = plsc.bitcast(b, jnp.int32)
                            lo = plsc.bitcast(w << 16, jnp.float32)
                            hi = plsc.bitcast(w & himask, jnp.float32)
                            new[2 * c] = new[2 * c] + lo
                            new[2 * c + 1] = new[2 * c + 1] + hi
                    return tuple(new)

                accs = lax.fori_loop(
                    0, seq // 8, red8,
                    tuple(jnp.zeros((16,), jnp.float32)
                          for _ in range(nacc)))
                for k in range(nacc):
                    acc_v[row, pl.ds(16 * k, 16)] = accs[k] * scale

                nxt = row + _NBUF

                @pl.when(nxt < b_per_w)
                def _():
                    issue(nxt, s)
            return carry

        lax.fori_loop(0, b_per_w // _NBUF, outer, 0)
        pltpu.sync_copy(acc_v, out_hbm.at[pl.ds(base, b_per_w)])

    return pool


def _mlp_body(p_ref, w1t_ref, b1_ref, w2t_ref, b2_ref, o_ref):
    h = jnp.dot(p_ref[...], w1t_ref[...],
                preferred_element_type=jnp.float32) + b1_ref[...]
    h = jnp.maximum(h, 0.0)
    o_ref[...] = jnp.dot(h, w2t_ref[...],
                         preferred_element_type=jnp.float32) + b2_ref[...]


def kernel(x, emb, W1, b1, W2, b2):
    batch, seq = x.shape
    out_d = W2.shape[0]
    vocab, embd = emb.shape
    x = x.astype(jnp.int32)

    out_cols = 128
    tr = pl.pallas_call(
        _tr_body,
        grid=((vocab + _TR_CHUNK - 1) // _TR_CHUNK,),
        in_specs=[pl.BlockSpec((embd, _TR_CHUNK), lambda i: (0, i))],
        out_specs=pl.BlockSpec((_TR_CHUNK, out_cols), lambda i: (i, 0)),
        out_shape=jax.ShapeDtypeStruct((vocab, out_cols), jnp.bfloat16),
    )
    emb_lin = tr(emb.T)

    pool = _make_pool(batch, seq, embd, out_cols)
    pooled = pool(x, emb_lin)

    sigma = []
    for g in range(embd // 32):
        sigma += [32 * g + 2 * j for j in range(16)]
        sigma += [32 * g + 2 * j + 1 for j in range(16)]
    w1tp = jnp.transpose(W1)[jnp.array(sigma), :]

    mlp = pl.pallas_call(
        _mlp_body,
        out_shape=jax.ShapeDtypeStruct((batch, out_d), jnp.float32),
    )
    return mlp(pooled, w1tp, b1[None, :], W2.T, b2[None, :])


# speedup vs baseline: 1.6206x; 1.6206x over previous
import functools
import jax
import jax.numpy as jnp
from jax import lax
from jax.experimental import pallas as pl
from jax.experimental.pallas import tpu as pltpu
from jax.experimental.pallas import tpu_sc as plsc

_NC = 2
_NS = 16
_NW = _NC * _NS
_NBUF = 4
_TR_CHUNK = 4096


def _tr_body(in_ref, out_ref):
    t = jnp.swapaxes(in_ref[...], 0, 1)
    t3 = t.reshape(t.shape[0] // 2, 2, t.shape[1])
    out_ref[...] = jnp.concatenate(
        [t3[:, 0, :], t3[:, 1, :]], axis=-1)


def _make_pool(batch, seq, emb, tw, nbuf):
    assert batch % _NW == 0
    b_per_w = batch // _NW
    assert b_per_w % nbuf == 0
    c0 = 128 if seq > 128 else seq
    c1 = seq - c0
    assert c0 % 8 == 0 and c1 % 8 == 0 and 0 < c1 <= 128
    mesh = plsc.VectorSubcoreMesh(
        core_axis_name="c", subcore_axis_name="s",
        num_cores=_NC, num_subcores=_NS)

    @functools.partial(
        pl.kernel,
        out_type=jax.ShapeDtypeStruct((batch, emb), jnp.float32),
        mesh=mesh,
        scratch_types=[
            pltpu.VMEM((b_per_w, seq), jnp.int32),
            pltpu.VMEM((nbuf, seq, tw), jnp.float32),
            pltpu.VMEM((b_per_w, emb), jnp.float32),
        ] + [pltpu.SemaphoreType.DMA] * nbuf,
        compiler_params=pltpu.CompilerParams(use_tc_tiling_on_sc=False),
    )
    def pool(x_hbm, emb_hbm, out_hbm, idx_v, rows_v, acc_v, *sems):
        wid = lax.axis_index("s") * _NC + lax.axis_index("c")
        base = wid * b_per_w
        pltpu.sync_copy(x_hbm.at[pl.ds(base, b_per_w)], idx_v)
        scale = jnp.float32(1.0 / seq)

        def issue(row, slot):
            pltpu.async_copy(
                emb_hbm.at[idx_v.at[row, pl.ds(0, c0)]],
                rows_v.at[slot, pl.ds(0, c0)], sems[slot])
            pltpu.async_copy(
                emb_hbm.at[idx_v.at[row, pl.ds(c0, c1)]],
                rows_v.at[slot, pl.ds(c0, c1)], sems[slot])

        def wait_slot(slot):
            pltpu.make_async_copy(
                emb_hbm.at[idx_v.at[0, pl.ds(0, c0)]],
                rows_v.at[slot, pl.ds(0, c0)], sems[slot]).wait()
            pltpu.make_async_copy(
                emb_hbm.at[idx_v.at[0, pl.ds(c0, c1)]],
                rows_v.at[slot, pl.ds(c0, c1)], sems[slot]).wait()

        for s in range(nbuf):
            issue(jnp.int32(s), s)

        nch = emb // 16

        def outer(g_idx, carry):
            g = g_idx * nbuf
            for s in range(nbuf):
                row = g + s
                wait_slot(s)

                def red8(i, accs):
                    r0 = i * 8
                    new = list(accs)
                    for r in range(8):
                        for c in range(nch):
                            new[c] = new[c] + rows_v[s, r0 + r,
                                                     pl.ds(16 * c, 16)]
                    return tuple(new)

                accs = lax.fori_loop(
                    0, seq // 8, red8,
                    tuple(jnp.zeros((16,), jnp.float32)
                          for _ in range(nch)))
                for c in range(nch):
                    acc_v[row, pl.ds(16 * c, 16)] = accs[c] * scale

                nxt = row + nbuf

                @pl.when(nxt < b_per_w)
                def _():
                    issue(nxt, s)
            return carry

        lax.fori_loop(0, b_per_w // nbuf, outer, 0)
        pltpu.sync_copy(acc_v, out_hbm.at[pl.ds(base, b_per_w)])

    return pool


def _mlp_body(p_ref, w1t_ref, b1_ref, w2t_ref, b2_ref, o_ref):
    h = jnp.dot(p_ref[...], w1t_ref[...],
                preferred_element_type=jnp.float32) + b1_ref[...]
    h = jnp.maximum(h, 0.0)
    o_ref[...] = jnp.dot(h, w2t_ref[...],
                         preferred_element_type=jnp.float32) + b2_ref[...]


def kernel(x, emb, W1, b1, W2, b2):
    batch, seq = x.shape
    out_d = W2.shape[0]
    vocab, embd = emb.shape
    x = x.astype(jnp.int32)

    emb_p = jnp.pad(emb, ((0, 0), (0, 128 - embd)))

    pool = _make_pool(batch, seq, embd, 128, 2)
    pooled = pool(x, emb_p)

    mlp = pl.pallas_call(
        _mlp_body,
        out_shape=jax.ShapeDtypeStruct((batch, out_d), jnp.float32),
    )
    return mlp(pooled, W1.T, b1[None, :], W2.T, b2[None, :])


# speedup vs baseline: 2.4165x; 1.4911x over previous
import functools
import jax
import jax.numpy as jnp
from jax import lax
from jax.experimental import pallas as pl
from jax.experimental.pallas import tpu as pltpu
from jax.experimental.pallas import tpu_sc as plsc

_NC = 2
_NS = 16
_NW = _NC * _NS
_NBUF = 4
_TR_CHUNK = 8192


def _tr_body(in_ref, out_ref):
    t = jnp.swapaxes(in_ref[...], 0, 1)
    t3 = t.reshape(t.shape[0] // 2, 2, t.shape[1])
    out_ref[...] = jnp.concatenate(
        [t3[:, 0, :], t3[:, 1, :]], axis=-1)


def _make_pool(batch, seq, emb):
    assert batch % _NW == 0
    b_per_w = batch // _NW
    c0 = 128 if seq > 128 else seq
    c1 = seq - c0
    assert c0 % 8 == 0 and c1 % 8 == 0 and 0 < c1 <= 128
    mesh = plsc.VectorSubcoreMesh(
        core_axis_name="c", subcore_axis_name="s",
        num_cores=_NC, num_subcores=_NS)

    @functools.partial(
        pl.kernel,
        out_type=jax.ShapeDtypeStruct((batch, emb), jnp.float32),
        mesh=mesh,
        scratch_types=[
            pltpu.VMEM((b_per_w, seq), jnp.int32),
            pltpu.VMEM((_NBUF, seq, emb), jnp.float32),
            pltpu.VMEM((b_per_w, emb), jnp.float32),
        ] + [pltpu.SemaphoreType.DMA] * _NBUF,
        compiler_params=pltpu.CompilerParams(use_tc_tiling_on_sc=False),
    )
    def pool(x_hbm, emb_hbm, out_hbm, idx_v, rows_v, acc_v, *sems):
        wid = lax.axis_index("s") * _NC + lax.axis_index("c")
        base = wid * b_per_w
        pltpu.sync_copy(x_hbm.at[pl.ds(base, b_per_w)], idx_v)
        scale = jnp.float32(1.0 / seq)

        def issue(row, slot):
            pltpu.async_copy(
                emb_hbm.at[idx_v.at[row, pl.ds(0, c0)]],
                rows_v.at[slot, pl.ds(0, c0)], sems[slot])
            pltpu.async_copy(
                emb_hbm.at[idx_v.at[row, pl.ds(c0, c1)]],
                rows_v.at[slot, pl.ds(c0, c1)], sems[slot])

        def wait_slot(slot):
            pltpu.make_async_copy(
                emb_hbm.at[idx_v.at[0, pl.ds(0, c0)]],
                rows_v.at[slot, pl.ds(0, c0)], sems[slot]).wait()
            pltpu.make_async_copy(
                emb_hbm.at[idx_v.at[0, pl.ds(c0, c1)]],
                rows_v.at[slot, pl.ds(c0, c1)], sems[slot]).wait()

        for s in range(_NBUF):
            issue(jnp.int32(s), s)

        nch = emb // 16

        def outer(g_idx, carry):
            g = g_idx * _NBUF
            for s in range(_NBUF):
                row = g + s
                wait_slot(s)

                def red8(i, accs):
                    r0 = i * 8
                    new = list(accs)
                    for r in range(8):
                        for c in range(nch):
                            new[c] = new[c] + rows_v[s, r0 + r,
                                                     pl.ds(16 * c, 16)]
                    return tuple(new)

                accs = lax.fori_loop(
                    0, seq // 8, red8,
                    tuple(jnp.zeros((16,), jnp.float32)
                          for _ in range(nch)))
                for c in range(nch):
                    acc_v[row, pl.ds(16 * c, 16)] = accs[c] * scale

                nxt = row + _NBUF

                @pl.when(nxt < b_per_w)
                def _():
                    issue(nxt, s)
            return carry

        lax.fori_loop(0, b_per_w // _NBUF, outer, 0)
        pltpu.sync_copy(acc_v, out_hbm.at[pl.ds(base, b_per_w)])

    return pool


def _mlp_body(p_ref, w1t_ref, b1_ref, w2t_ref, b2_ref, o_ref):
    h = jnp.dot(p_ref[...], w1t_ref[...],
                preferred_element_type=jnp.float32) + b1_ref[...]
    h = jnp.maximum(h, 0.0)
    o_ref[...] = jnp.dot(h, w2t_ref[...],
                         preferred_element_type=jnp.float32) + b2_ref[...]


def kernel(x, emb, W1, b1, W2, b2):
    batch, seq = x.shape
    out_d = W2.shape[0]
    vocab, embd = emb.shape
    x = x.astype(jnp.int32)

    out_cols = 128
    rows_per_chunk = _TR_CHUNK * embd // out_cols
    tr = pl.pallas_call(
        _tr_body,
        grid=((vocab + _TR_CHUNK - 1) // _TR_CHUNK,),
        in_specs=[pl.BlockSpec((embd, _TR_CHUNK), lambda i: (0, i))],
        out_specs=pl.BlockSpec((rows_per_chunk, out_cols), lambda i: (i, 0)),
        out_shape=jax.ShapeDtypeStruct(
            (vocab * embd // out_cols, out_cols), jnp.float32),
    )
    emb_flat = tr(emb.T)
    emb_lin = emb_flat.reshape(vocab, embd)

    pool = _make_pool(batch, seq, embd)
    pooled = pool(x, emb_lin)

    mlp = pl.pallas_call(
        _mlp_body,
        out_shape=jax.ShapeDtypeStruct((batch, out_d), jnp.float32),
    )
    return mlp(pooled, W1.T, b1[None, :], W2.T, b2[None, :])


# speedup vs baseline: 2.4351x; 1.0077x over previous
import functools
import jax
import jax.numpy as jnp
from jax import lax
from jax.experimental import pallas as pl
from jax.experimental.pallas import tpu as pltpu
from jax.experimental.pallas import tpu_sc as plsc

_NC = 2
_NS = 16
_NW = _NC * _NS
_NBUF = 4
_TR_CHUNK = 16384


def _tr_body(in_ref, out_ref):
    t = jnp.swapaxes(in_ref[...], 0, 1)
    t3 = t.reshape(t.shape[0] // 2, 2, t.shape[1])
    out_ref[...] = jnp.concatenate(
        [t3[:, 0, :], t3[:, 1, :]], axis=-1)


def _make_pool(batch, seq, emb):
    assert batch % _NW == 0
    b_per_w = batch // _NW
    c0 = 128 if seq > 128 else seq
    c1 = seq - c0
    assert c0 % 8 == 0 and c1 % 8 == 0 and 0 < c1 <= 128
    mesh = plsc.VectorSubcoreMesh(
        core_axis_name="c", subcore_axis_name="s",
        num_cores=_NC, num_subcores=_NS)

    @functools.partial(
        pl.kernel,
        out_type=jax.ShapeDtypeStruct((batch, emb), jnp.float32),
        mesh=mesh,
        scratch_types=[
            pltpu.VMEM((b_per_w, seq), jnp.int32),
            pltpu.VMEM((_NBUF, seq, emb), jnp.float32),
            pltpu.VMEM((b_per_w, emb), jnp.float32),
        ] + [pltpu.SemaphoreType.DMA] * _NBUF,
        compiler_params=pltpu.CompilerParams(use_tc_tiling_on_sc=False),
    )
    def pool(x_hbm, emb_hbm, out_hbm, idx_v, rows_v, acc_v, *sems):
        wid = lax.axis_index("s") * _NC + lax.axis_index("c")
        base = wid * b_per_w
        pltpu.sync_copy(x_hbm.at[pl.ds(base, b_per_w)], idx_v)
        scale = jnp.float32(1.0 / seq)

        def issue(row, slot):
            pltpu.async_copy(
                emb_hbm.at[idx_v.at[row, pl.ds(0, c0)]],
                rows_v.at[slot, pl.ds(0, c0)], sems[slot])
            pltpu.async_copy(
                emb_hbm.at[idx_v.at[row, pl.ds(c0, c1)]],
                rows_v.at[slot, pl.ds(c0, c1)], sems[slot])

        def wait_slot(slot):
            pltpu.make_async_copy(
                emb_hbm.at[idx_v.at[0, pl.ds(0, c0)]],
                rows_v.at[slot, pl.ds(0, c0)], sems[slot]).wait()
            pltpu.make_async_copy(
                emb_hbm.at[idx_v.at[0, pl.ds(c0, c1)]],
                rows_v.at[slot, pl.ds(c0, c1)], sems[slot]).wait()

        for s in range(_NBUF):
            issue(jnp.int32(s), s)

        nch = emb // 16

        def outer(g_idx, carry):
            g = g_idx * _NBUF
            for s in range(_NBUF):
                row = g + s
                wait_slot(s)

                def red8(i, accs):
                    r0 = i * 8
                    new = list(accs)
                    for r in range(8):
                        for c in range(nch):
                            new[c] = new[c] + rows_v[s, r0 + r,
                                                     pl.ds(16 * c, 16)]
                    return tuple(new)

                accs = lax.fori_loop(
                    0, seq // 8, red8,
                    tuple(jnp.zeros((16,), jnp.float32)
                          for _ in range(nch)))
                for c in range(nch):
                    acc_v[row, pl.ds(16 * c, 16)] = accs[c] * scale

                nxt = row + _NBUF

                @pl.when(nxt < b_per_w)
                def _():
                    issue(nxt, s)
            return carry

        lax.fori_loop(0, b_per_w // _NBUF, outer, 0)
        pltpu.sync_copy(acc_v, out_hbm.at[pl.ds(base, b_per_w)])

    return pool


def _mlp_body(p_ref, w1t_ref, b1_ref, w2t_ref, b2_ref, o_ref):
    h = jnp.dot(p_ref[...], w1t_ref[...],
                preferred_element_type=jnp.float32) + b1_ref[...]
    h = jnp.maximum(h, 0.0)
    o_ref[...] = jnp.dot(h, w2t_ref[...],
                         preferred_element_type=jnp.float32) + b2_ref[...]


def kernel(x, emb, W1, b1, W2, b2):
    batch, seq = x.shape
    out_d = W2.shape[0]
    vocab, embd = emb.shape
    x = x.astype(jnp.int32)

    out_cols = 128
    rows_per_chunk = _TR_CHUNK * embd // out_cols
    tr = pl.pallas_call(
        _tr_body,
        grid=((vocab + _TR_CHUNK - 1) // _TR_CHUNK,),
        in_specs=[pl.BlockSpec((embd, _TR_CHUNK), lambda i: (0, i))],
        out_specs=pl.BlockSpec((rows_per_chunk, out_cols), lambda i: (i, 0)),
        out_shape=jax.ShapeDtypeStruct(
            (vocab * embd // out_cols, out_cols), jnp.float32),
    )
    emb_flat = tr(emb.T)
    emb_lin = emb_flat.reshape(vocab, embd)

    pool = _make_pool(batch, seq, embd)
    pooled = pool(x, emb_lin)

    mlp = pl.pallas_call(
        _mlp_body,
        out_shape=jax.ShapeDtypeStruct((batch, out_d), jnp.float32),
    )
    return mlp(pooled, W1.T, b1[None, :], W2.T, b2[None, :])
